# baseline (speedup 1.0000x reference)
import functools

import jax
import jax.numpy as jnp
from jax import lax
from jax.experimental import pallas as pl
from jax.experimental.pallas import tpu as pltpu
from jax.experimental.pallas import tpu_sc as plsc

EMBED_DIM = 32
BATCH = 16384
NUM_ROWS = 100000

_NUM_CORES = 2
_NUM_SUBCORES = 16
_CHUNK = 2048
_NCHUNK = BATCH // _CHUNK
_NBUF = 3


@functools.partial(
    pl.kernel,
    mesh=plsc.VectorSubcoreMesh(core_axis_name="c", subcore_axis_name="s"),
    out_type=jax.ShapeDtypeStruct((EMBED_DIM, BATCH), jnp.float32),
    scratch_types=[
        pltpu.VMEM((NUM_ROWS,), jnp.float32),
        pltpu.VMEM((_CHUNK,), jnp.int32),
        pltpu.VMEM((_CHUNK,), jnp.int32),
        pltpu.VMEM((_CHUNK,), jnp.int32),
        pltpu.VMEM((BATCH,), jnp.float32),
        pltpu.SemaphoreType.DMA,
        pltpu.SemaphoreType.DMA,
        pltpu.SemaphoreType.DMA,
        pltpu.SemaphoreType.DMA,
        pltpu.SemaphoreType.DMA,
    ],
    compiler_params=pltpu.CompilerParams(needs_layout_passes=False),
)
def _gather_kernel(idx_hbm, tab_hbm, out_hbm, col_v, idx_v0, idx_v1, idx_v2,
                   row_v, csem, isem0, isem1, isem2, wsem):
    d = lax.axis_index("s") * _NUM_CORES + lax.axis_index("c")
    col_cp = pltpu.async_copy(tab_hbm.at[d, :], col_v, csem)

    ibufs = [idx_v0, idx_v1, idx_v2]
    isems = [isem0, isem1, isem2]

    def fetch(c):
        return pltpu.async_copy(
            idx_hbm.at[pl.ds(c * _CHUNK, _CHUNK)], ibufs[c % _NBUF], isems[c % _NBUF])

    pending = {c: fetch(c) for c in range(min(_NBUF, _NCHUNK))}
    col_cp.wait()

    write_cps = []
    for c in range(_NCHUNK):
        pending[c].wait()
        buf = ibufs[c % _NBUF]

        @plsc.parallel_loop(0, _CHUNK, step=16, unroll=8)
        def gather_body(i):
            iv = buf[pl.ds(i, 16)]
            row_v[pl.ds(c * _CHUNK + i, 16)] = plsc.load_gather(col_v, [iv])

        if c + _NBUF < _NCHUNK:
            pending[c + _NBUF] = fetch(c + _NBUF)
        write_cps.append(pltpu.async_copy(
            row_v.at[pl.ds(c * _CHUNK, _CHUNK)],
            out_hbm.at[d, pl.ds(c * _CHUNK, _CHUNK)], wsem))
    for cp in write_cps:
        cp.wait()


def kernel(platform_ids, table):
    out_t = _gather_kernel(platform_ids.astype(jnp.int32), table.T)
    return out_t.T


# speedup vs baseline: 1.2057x; 1.2057x over previous
import functools

import jax
import jax.numpy as jnp
from jax import lax
from jax.experimental import pallas as pl
from jax.experimental.pallas import tpu as pltpu
from jax.experimental.pallas import tpu_sc as plsc

EMBED_DIM = 32
BATCH = 16384
NUM_ROWS = 100000

_NUM_CORES = 2
_NUM_SUBCORES = 16
_CHUNK = 4096
_NCHUNK = BATCH // _CHUNK
_NBUF = 3


@functools.partial(
    pl.kernel,
    mesh=plsc.VectorSubcoreMesh(core_axis_name="c", subcore_axis_name="s"),
    out_type=jax.ShapeDtypeStruct((EMBED_DIM, BATCH), jnp.float32),
    scratch_types=[
        pltpu.VMEM((NUM_ROWS,), jnp.float32),
        pltpu.VMEM((_CHUNK,), jnp.int32),
        pltpu.VMEM((_CHUNK,), jnp.int32),
        pltpu.VMEM((_CHUNK,), jnp.int32),
        pltpu.VMEM((BATCH,), jnp.float32),
        pltpu.MemorySpace.VMEM_SHARED((BATCH,), jnp.int32),
        pltpu.SemaphoreType.DMA,
        pltpu.SemaphoreType.DMA,
        pltpu.SemaphoreType.DMA,
        pltpu.SemaphoreType.DMA,
        pltpu.SemaphoreType.DMA,
    ],
    compiler_params=pltpu.CompilerParams(needs_layout_passes=False),
)
def _gather_kernel(idx_hbm, tab_hbm, out_hbm, col_v, idx_v0, idx_v1, idx_v2,
                   row_v, sidx, csem, isem0, isem1, isem2, wsem):
    s = lax.axis_index("s")
    d = s * _NUM_CORES + lax.axis_index("c")
    col_cp = pltpu.async_copy(tab_hbm.at[d, :], col_v, csem)

    @pl.when(s == 0)
    def _stage_ids():
        pltpu.sync_copy(idx_hbm, sidx)

    plsc.subcore_barrier()

    ibufs = [idx_v0, idx_v1, idx_v2]
    isems = [isem0, isem1, isem2]

    def fetch(c):
        return pltpu.async_copy(
            sidx.at[pl.ds(c * _CHUNK, _CHUNK)], ibufs[c % _NBUF], isems[c % _NBUF])

    pending = {c: fetch(c) for c in range(min(_NBUF, _NCHUNK))}
    col_cp.wait()

    write_cps = []
    for c in range(_NCHUNK):
        pending[c].wait()
        buf = ibufs[c % _NBUF]

        @plsc.parallel_loop(0, _CHUNK, step=16, unroll=8)
        def gather_body(i):
            iv = buf[pl.ds(i, 16)]
            row_v[pl.ds(c * _CHUNK + i, 16)] = plsc.load_gather(col_v, [iv])

        if c + _NBUF < _NCHUNK:
            pending[c + _NBUF] = fetch(c + _NBUF)
        write_cps.append(pltpu.async_copy(
            row_v.at[pl.ds(c * _CHUNK, _CHUNK)],
            out_hbm.at[d, pl.ds(c * _CHUNK, _CHUNK)], wsem))
    for cp in write_cps:
        cp.wait()


def kernel(platform_ids, table):
    out_t = _gather_kernel(platform_ids.astype(jnp.int32), table.T)
    return out_t.T
